# speedup vs baseline: 20.8833x; 20.8833x over previous
import functools

import jax
import jax.numpy as jnp
from jax.experimental import pallas as pl
from jax.experimental.pallas import tpu as pltpu

_K = 512
_ROW_BLOCK = 16


def _topk_mask_body(x_ref, o_ref):
    x = x_ref[...]
    u = jax.lax.bitcast_convert_type(x, jnp.uint32)
    neg = (u >> jnp.uint32(31)).astype(jnp.uint32)
    key = u ^ (neg * jnp.uint32(0x7FFFFFFF) + jnp.uint32(0x80000000))

    t = jnp.zeros((x.shape[0], 1), dtype=jnp.uint32)
    for b in range(31, -1, -1):
        cand = t | jnp.uint32(1 << b)
        cnt = jnp.sum((key >= cand).astype(jnp.int32), axis=1, keepdims=True)
        t = jnp.where(cnt >= _K, cand, t)

    o_ref[...] = jnp.where(key >= t, x, jnp.float32(0.0))


def kernel(x):
    rows, cols = x.shape
    grid = (rows // _ROW_BLOCK,)
    return pl.pallas_call(
        _topk_mask_body,
        grid=grid,
        in_specs=[pl.BlockSpec((_ROW_BLOCK, cols), lambda i: (i, 0))],
        out_specs=pl.BlockSpec((_ROW_BLOCK, cols), lambda i: (i, 0)),
        out_shape=jax.ShapeDtypeStruct((rows, cols), x.dtype),
    )(x)
